# Initial kernel scaffold; baseline (speedup 1.0000x reference)
import jax
import jax.numpy as jnp
from jax.experimental import pallas as pl


def kernel(hidden_states, embedding):
    raise NotImplementedError("write your pallas kernel here")


# speedup vs baseline: 1.1368x; 1.1368x over previous
import functools

import jax
import jax.numpy as jnp
from jax import lax
from jax.experimental import pallas as pl
from jax.experimental.pallas import tpu as pltpu
from jax.experimental.pallas import tpu_sc as plsc

M = 8192
N = 8192
K = 256
TM = 256

_NC = 2
_NS = 16
_NW = _NC * _NS
_BPW = M // _NW
_CHUNK = 128
_NCHUNK = _BPW // _CHUNK


def _argmin_body(hn2_ref, en2_ref, h_ref, e_ref, out_ref):
    scores = hn2_ref[...] + en2_ref[...] - 2.0 * lax.dot_general(
        h_ref[...], e_ref[...], (((1,), (1,)), ((), ())),
        preferred_element_type=jnp.float32)
    m = jnp.min(scores, axis=1, keepdims=True)
    iota = lax.broadcasted_iota(jnp.int32, scores.shape, 1)
    idx = jnp.min(jnp.where(scores == m, iota, jnp.int32(N)), axis=1)
    out_ref[...] = idx.reshape(1, 1, TM)


def _argmin_call(hn2, en2, hf, emb):
    grid = (M // TM,)
    return pl.pallas_call(
        _argmin_body,
        grid=grid,
        in_specs=[
            pl.BlockSpec((TM, 1), lambda i: (i, 0)),
            pl.BlockSpec((1, N), lambda i: (0, 0)),
            pl.BlockSpec((TM, K), lambda i: (i, 0)),
            pl.BlockSpec((N, K), lambda i: (0, 0)),
        ],
        out_specs=pl.BlockSpec((1, 1, TM), lambda i: (i, 0, 0)),
        out_shape=jax.ShapeDtypeStruct((M // TM, 1, TM), jnp.int32),
    )(hn2, en2, hf, emb)


def _gather_body(table_hbm, idx_hbm, out_hbm, idx_v, rows_v, sem):
    wid = lax.axis_index("s") * _NC + lax.axis_index("c")
    base = wid * _BPW
    pltpu.sync_copy(idx_hbm.at[wid], idx_v)
    copies = [
        pltpu.async_copy(table_hbm.at[idx_v.at[j]],
                         rows_v.at[pl.ds(j * _CHUNK, _CHUNK)], sem)
        for j in range(_NCHUNK)
    ]
    for c in copies:
        c.wait()
    pltpu.sync_copy(rows_v, out_hbm.at[pl.ds(base, _BPW)])


@functools.cache
def _gather_call():
    return functools.partial(
        pl.kernel,
        mesh=plsc.VectorSubcoreMesh(core_axis_name="c", subcore_axis_name="s",
                                    num_cores=_NC, num_subcores=_NS),
        out_type=jax.ShapeDtypeStruct((M, K), jnp.float32),
        scratch_types=[
            pltpu.VMEM((_NCHUNK, _CHUNK), jnp.int32),
            pltpu.VMEM((_BPW, K), jnp.float32),
            pltpu.SemaphoreType.DMA,
        ],
    )(_gather_body)


def kernel(hidden_states, embedding):
    hf = hidden_states.reshape(-1, K)
    hn2 = jnp.sum(hf ** 2, axis=1, keepdims=True)
    en2 = jnp.sum(embedding ** 2, axis=1).reshape(1, N)
    idx = _argmin_call(hn2, en2, hf, embedding).reshape(M)
    z_q = _gather_call()(embedding, idx.reshape(_NW, _NCHUNK, _CHUNK))
    return (z_q.reshape(hidden_states.shape),
            idx.reshape(hidden_states.shape[0], -1))
